# baseline (speedup 1.0000x reference)
import jax
import jax.numpy as jnp
from jax.experimental import pallas as pl
from jax.experimental.pallas import tpu as pltpu

_BV = 640
_K = 5


def _hidden_body(x_ref, w1_ref, b1_ref, h_ref):
    for k in range(4):
        h = jax.lax.dot_general(
            x_ref[...], w1_ref[k],
            dimension_numbers=(((1,), (1,)), ((), ())),
            preferred_element_type=jnp.float32,
        ) + b1_ref[k]
        h_ref[k] = h * jax.nn.sigmoid(h)


def _stream_body(h_ref, w2_ref, o0, o1, o2, o3, buf_ref, sem_ref):
    j = pl.program_id(0)
    nsteps = pl.num_programs(0)

    def _issue(step, slot):
        for k in range(4):
            pltpu.make_async_copy(
                w2_ref.at[k, pl.ds(step * _BV, _BV), :],
                buf_ref.at[slot, k],
                sem_ref.at[slot, k],
            ).start()

    @pl.when(j == 0)
    def _prologue():
        for s in range(_K):
            _issue(s, s)

    slot = jax.lax.rem(j, _K)
    nxt = j + _K
    for k, o in enumerate((o0, o1, o2, o3)):
        pltpu.make_async_copy(
            w2_ref.at[k, pl.ds(j * _BV, _BV), :],
            buf_ref.at[slot, k],
            sem_ref.at[slot, k],
        ).wait()
        o[...] = jax.lax.dot_general(
            h_ref[k], buf_ref[slot, k],
            dimension_numbers=(((1,), (1,)), ((), ())),
            preferred_element_type=jnp.float32,
        )

        @pl.when(nxt < nsteps)
        def _refill():
            pltpu.make_async_copy(
                w2_ref.at[k, pl.ds(nxt * _BV, _BV), :],
                buf_ref.at[slot, k],
                sem_ref.at[slot, k],
            ).start()


def kernel(hidden_states, W1, b1, W2):
    B, S, H = hidden_states.shape
    NH, V, _ = W2.shape
    x = hidden_states.reshape(B * S, H)

    h = pl.pallas_call(
        _hidden_body,
        out_shape=jax.ShapeDtypeStruct((NH, B * S, H), jnp.float32),
    )(x, W1, b1)

    outs = pl.pallas_call(
        _stream_body,
        grid=(V // _BV,),
        in_specs=[
            pl.BlockSpec((NH, B * S, H), lambda j: (0, 0, 0)),
            pl.BlockSpec(memory_space=pltpu.MemorySpace.HBM),
        ],
        out_specs=[pl.BlockSpec((B * S, _BV), lambda j: (0, j))
                   for _ in range(NH)],
        out_shape=[jax.ShapeDtypeStruct((B * S, V), jnp.float32)
                   for _ in range(NH)],
        scratch_shapes=[
            pltpu.VMEM((_K, NH, _BV, H), jnp.float32),
            pltpu.SemaphoreType.DMA((_K, NH)),
        ],
        compiler_params=pltpu.CompilerParams(
            dimension_semantics=("arbitrary",),
            vmem_limit_bytes=60 * 1024 * 1024,
        ),
    )(h, W2)

    return tuple(o.reshape(B, S, V) for o in outs)


# speedup vs baseline: 1.0045x; 1.0045x over previous
import jax
import jax.numpy as jnp
from jax.experimental import pallas as pl
from jax.experimental.pallas import tpu as pltpu

_BV = 640
_K = 4


def _medusa_body(x_ref, b1_ref, w1_hbm, w2_ref, o0, o1, o2, o3,
                 h_ref, w1_ref, buf_ref, sem_ref, w1_sem):
    j = pl.program_id(0)
    nsteps = pl.num_programs(0)

    def _issue(step, slot):
        for k in range(4):
            pltpu.make_async_copy(
                w2_ref.at[k, pl.ds(step * _BV, _BV), :],
                buf_ref.at[slot, k],
                sem_ref.at[slot, k],
            ).start()

    @pl.when(j == 0)
    def _prologue():
        pltpu.make_async_copy(w1_hbm, w1_ref, w1_sem).start()
        for s in range(_K):
            _issue(s, s)
        pltpu.make_async_copy(w1_hbm, w1_ref, w1_sem).wait()
        for k in range(4):
            h = jax.lax.dot_general(
                x_ref[...], w1_ref[k],
                dimension_numbers=(((1,), (1,)), ((), ())),
                preferred_element_type=jnp.float32,
            ) + b1_ref[k]
            h_ref[k] = h * jax.nn.sigmoid(h)

    slot = jax.lax.rem(j, _K)
    nxt = j + _K
    for k, o in enumerate((o0, o1, o2, o3)):
        pltpu.make_async_copy(
            w2_ref.at[k, pl.ds(j * _BV, _BV), :],
            buf_ref.at[slot, k],
            sem_ref.at[slot, k],
        ).wait()
        o[...] = jax.lax.dot_general(
            h_ref[k], buf_ref[slot, k],
            dimension_numbers=(((1,), (1,)), ((), ())),
            preferred_element_type=jnp.float32,
        )

        @pl.when(nxt < nsteps)
        def _refill():
            pltpu.make_async_copy(
                w2_ref.at[k, pl.ds(nxt * _BV, _BV), :],
                buf_ref.at[slot, k],
                sem_ref.at[slot, k],
            ).start()


def kernel(hidden_states, W1, b1, W2):
    B, S, H = hidden_states.shape
    NH, V, _ = W2.shape
    x = hidden_states.reshape(B * S, H)

    outs = pl.pallas_call(
        _medusa_body,
        grid=(V // _BV,),
        in_specs=[
            pl.BlockSpec((B * S, H), lambda j: (0, 0)),
            pl.BlockSpec((NH, H), lambda j: (0, 0)),
            pl.BlockSpec(memory_space=pltpu.MemorySpace.HBM),
            pl.BlockSpec(memory_space=pltpu.MemorySpace.HBM),
        ],
        out_specs=[pl.BlockSpec((B * S, _BV), lambda j: (0, j))
                   for _ in range(NH)],
        out_shape=[jax.ShapeDtypeStruct((B * S, V), jnp.float32)
                   for _ in range(NH)],
        scratch_shapes=[
            pltpu.VMEM((NH, B * S, H), jnp.float32),
            pltpu.VMEM((NH, H, H), jnp.float32),
            pltpu.VMEM((_K, NH, _BV, H), jnp.float32),
            pltpu.SemaphoreType.DMA((_K, NH)),
            pltpu.SemaphoreType.DMA,
        ],
        compiler_params=pltpu.CompilerParams(
            dimension_semantics=("arbitrary",),
            vmem_limit_bytes=62 * 1024 * 1024,
        ),
    )(x, b1, W1, W2)

    return tuple(o.reshape(B, S, V) for o in outs)


# speedup vs baseline: 1.0192x; 1.0146x over previous
import jax
import jax.numpy as jnp
from jax.experimental import pallas as pl
from jax.experimental.pallas import tpu as pltpu

_BV = 640


def _medusa_body(x_ref, w1_ref, b1_ref, w2_ref, o0, o1, o2, o3, h_ref):
    j = pl.program_id(0)

    @pl.when(j == 0)
    def _compute_hidden():
        for k in range(4):
            h = jax.lax.dot_general(
                x_ref[...], w1_ref[k],
                dimension_numbers=(((1,), (1,)), ((), ())),
                preferred_element_type=jnp.float32,
            ) + b1_ref[k]
            h_ref[k] = h * jax.nn.sigmoid(h)

    for k, o in enumerate((o0, o1, o2, o3)):
        o[...] = jax.lax.dot_general(
            h_ref[k], w2_ref[k],
            dimension_numbers=(((1,), (1,)), ((), ())),
            preferred_element_type=jnp.float32,
        )


def kernel(hidden_states, W1, b1, W2):
    B, S, H = hidden_states.shape
    NH, V, _ = W2.shape
    x = hidden_states.reshape(B * S, H)

    outs = pl.pallas_call(
        _medusa_body,
        grid=(V // _BV,),
        in_specs=[
            pl.BlockSpec((B * S, H), lambda j: (0, 0)),
            pl.BlockSpec((NH, H, H), lambda j: (0, 0, 0)),
            pl.BlockSpec((NH, H), lambda j: (0, 0)),
            pl.BlockSpec((NH, _BV, H), lambda j: (0, j, 0)),
        ],
        out_specs=[pl.BlockSpec((B * S, _BV), lambda j: (0, j))
                   for _ in range(NH)],
        out_shape=[jax.ShapeDtypeStruct((B * S, V), jnp.float32)
                   for _ in range(NH)],
        scratch_shapes=[pltpu.VMEM((NH, B * S, H), jnp.float32)],
        compiler_params=pltpu.CompilerParams(
            dimension_semantics=("arbitrary",),
            vmem_limit_bytes=128 * 1024 * 1024,
        ),
    )(x, W1, b1, W2)

    return tuple(o.reshape(B, S, V) for o in outs)
